# Initial kernel scaffold; baseline (speedup 1.0000x reference)
import jax
import jax.numpy as jnp
from jax.experimental import pallas as pl


def kernel(embs, edge_index, rel_type, batch_size, weight1, root1, bias1, weight2, root2, bias2):
    raise NotImplementedError("write your pallas kernel here")


# speedup vs baseline: 8.2996x; 8.2996x over previous
import functools

import jax
import jax.numpy as jnp
from jax import lax
from jax.experimental import pallas as pl
from jax.experimental.pallas import tpu as pltpu
from jax.experimental.pallas import tpu_sc as plsc

N = 10000
E = 320000
D = 128
R = 8

NC = 2
NS = 16
L = 16
NPARTS = D // L
PPC = NPARTS // NC
SHARD = E // NS
EB = 128
NB = (SHARD + EB - 1) // EB
PAD = NB * EB
RN = R * N
DUMMY = RN
ACC_ROWS = RN + EB
TPR = RN // NS
ZROWS = 250
CH = 1024


def _precompute_aidx(dst_hbm, rel_hbm, dbuf, rbuf, aidx_st, t, lanes):
    base = t * SHARD

    def chunk(ci, carry):
        pltpu.sync_copy(dst_hbm.at[pl.ds(base + ci * CH, CH)], dbuf)
        pltpu.sync_copy(rel_hbm.at[pl.ds(base + ci * CH, CH)], rbuf)

        def vec(k, carry2):
            off = k * L
            d16 = dbuf[pl.ds(off, L)]
            r16 = rbuf[pl.ds(off, L)]
            aidx_st[pl.ds(ci * CH + off, L)] = r16 * N + d16
            return carry2

        lax.fori_loop(0, CH // L, vec, 0)
        return carry

    nfull = SHARD // CH
    lax.fori_loop(0, nfull, chunk, 0)
    tail = SHARD - nfull * CH
    pltpu.sync_copy(dst_hbm.at[pl.ds(base + nfull * CH, tail)],
                    dbuf.at[pl.ds(0, tail)])
    pltpu.sync_copy(rel_hbm.at[pl.ds(base + nfull * CH, tail)],
                    rbuf.at[pl.ds(0, tail)])

    def tailvec(k, carry2):
        off = k * L
        eid = nfull * CH + off + lanes
        valid = eid < SHARD
        d16 = dbuf[pl.ds(off, L)]
        r16 = rbuf[pl.ds(off, L)]
        aidx_st[pl.ds(nfull * CH + off, L)] = jnp.where(
            valid, r16 * N + d16, jnp.int32(DUMMY))
        return carry2

    lax.fori_loop(0, (PAD - nfull * CH) // L, tailvec, 0)


def _zero_acc(zbuf, acc, t):
    for i in range(TPR // ZROWS):
        pltpu.sync_copy(zbuf, acc.at[pl.ds(t * TPR + i * ZROWS, ZROWS)])


def _sc_parts_body(xp_hbm, src_hbm, dst_hbm, rel_hbm, zeros_hbm,
                   a_hbm,
                   src_st, aidx_st, dbuf, rbuf, sidx_v, aidx_v, rows_v, zbuf,
                   acc):
    c = lax.axis_index("c")
    t = lax.axis_index("s")
    lanes = jnp.arange(L, dtype=jnp.int32)
    pltpu.sync_copy(src_hbm.at[pl.ds(t * SHARD, SHARD)],
                    src_st.at[pl.ds(0, SHARD)])
    _precompute_aidx(dst_hbm, rel_hbm, dbuf, rbuf, aidx_st, t, lanes)
    pltpu.sync_copy(zeros_hbm, zbuf)

    def part_pass(p, carry):
        pglob = c * PPC + p
        _zero_acc(zbuf, acc, t)
        plsc.subcore_barrier()

        def edge_step(j, carry2):
            for k in range(EB // L):
                off = j * EB + k * L
                eid = off + lanes
                valid = eid < SHARD
                s16 = src_st[pl.ds(off, L)]
                sidx_v[pl.ds(k * L, L)] = jnp.where(valid, s16 + pglob * N,
                                                    jnp.int32(0))
                aidx_v[pl.ds(k * L, L)] = aidx_st[pl.ds(off, L)]
            pltpu.sync_copy(xp_hbm.at[sidx_v], rows_v)
            pltpu.sync_copy(rows_v, acc.at[aidx_v], add=True)
            return carry2

        lax.fori_loop(0, NB, edge_step, 0)
        plsc.subcore_barrier()
        r_own = t // 2
        n_own = (t % 2) * TPR
        pltpu.sync_copy(acc.at[pl.ds(t * TPR, TPR)],
                        a_hbm.at[pl.ds(n_own, TPR), r_own,
                                 pl.ds(pglob * L, L)])
        return carry

    lax.fori_loop(0, PPC, part_pass, 0)


def _sc_cnt_body(dst_hbm, rel_hbm, zeros_hbm,
                 cnt_hbm,
                 aidx_st, dbuf, rbuf, aidx_v, ones_v, zbuf, acc):
    c = lax.axis_index("c")
    t = lax.axis_index("s")
    lanes = jnp.arange(L, dtype=jnp.int32)
    _precompute_aidx(dst_hbm, rel_hbm, dbuf, rbuf, aidx_st, t, lanes)
    pltpu.sync_copy(zeros_hbm, zbuf)
    for k in range(EB // L):
        ones_v[pl.ds(k * L, L), :] = jnp.ones((L, L), jnp.float32)

    _zero_acc(zbuf, acc, t)
    plsc.subcore_barrier()

    def cnt_step(j, carry):
        for k in range(EB // L):
            aidx_v[pl.ds(k * L, L)] = aidx_st[pl.ds(j * EB + k * L, L)]
        pltpu.sync_copy(ones_v, acc.at[aidx_v], add=True)
        return carry

    half = (NB + 1) // 2
    lax.fori_loop(c * half, jnp.minimum(NB, (c + 1) * half), cnt_step, 0)
    plsc.subcore_barrier()
    pltpu.sync_copy(acc.at[pl.ds(t * TPR, TPR)],
                    cnt_hbm.at[c, pl.ds(t * TPR, TPR)])


_SC_MESH = dict(core_axis_name="c", subcore_axis_name="s")


def _make_sc_parts():
    return pl.kernel(
        _sc_parts_body,
        out_type=jax.ShapeDtypeStruct((N, R, D), jnp.float32),
        mesh=plsc.VectorSubcoreMesh(**_SC_MESH),
        scratch_types=[
            pltpu.VMEM((PAD,), jnp.int32),
            pltpu.VMEM((PAD,), jnp.int32),
            pltpu.VMEM((CH,), jnp.int32),
            pltpu.VMEM((CH,), jnp.int32),
            pltpu.VMEM((EB,), jnp.int32),
            pltpu.VMEM((EB,), jnp.int32),
            pltpu.VMEM((EB, L), jnp.float32),
            pltpu.VMEM((ZROWS, L), jnp.float32),
            pltpu.VMEM_SHARED((ACC_ROWS, L), jnp.float32),
        ],
        compiler_params=pltpu.CompilerParams(use_tc_tiling_on_sc=False),
    )


def _make_sc_cnt():
    return pl.kernel(
        _sc_cnt_body,
        out_type=jax.ShapeDtypeStruct((NC, RN, L), jnp.float32),
        mesh=plsc.VectorSubcoreMesh(**_SC_MESH),
        scratch_types=[
            pltpu.VMEM((PAD,), jnp.int32),
            pltpu.VMEM((CH,), jnp.int32),
            pltpu.VMEM((CH,), jnp.int32),
            pltpu.VMEM((EB,), jnp.int32),
            pltpu.VMEM((EB, L), jnp.float32),
            pltpu.VMEM((ZROWS, L), jnp.float32),
            pltpu.VMEM_SHARED((ACC_ROWS, L), jnp.float32),
        ],
        compiler_params=pltpu.CompilerParams(use_tc_tiling_on_sc=False),
    )


def _tc_body(apply_gelu, x_ref, ap_ref, cnt_ref, w_ref, root_ref, bias_ref, o_ref):
    x = x_ref[...]
    acc = jnp.dot(x, root_ref[...], preferred_element_type=jnp.float32)
    acc = acc + bias_ref[...]
    cnt = cnt_ref[:, :R] + cnt_ref[:, R:]
    inv = 1.0 / jnp.clip(cnt, 1.0, None)
    for r in range(R):
        a = ap_ref[:, r, :] * inv[:, r][:, None]
        acc = acc + jnp.dot(a, w_ref[r], preferred_element_type=jnp.float32)
    if apply_gelu:
        acc = 0.5 * acc * (1.0 + lax.erf(acc * (2.0 ** -0.5)))
    o_ref[...] = acc


NBLK = 1000


def _make_tc(apply_gelu):
    return pl.pallas_call(
        functools.partial(_tc_body, apply_gelu),
        grid=(N // NBLK,),
        in_specs=[
            pl.BlockSpec((NBLK, D), lambda i: (i, 0)),
            pl.BlockSpec((NBLK, R, D), lambda i: (i, 0, 0)),
            pl.BlockSpec((NBLK, NC * R), lambda i: (i, 0)),
            pl.BlockSpec((R, D, D), lambda i: (0, 0, 0)),
            pl.BlockSpec((D, D), lambda i: (0, 0)),
            pl.BlockSpec((1, D), lambda i: (0, 0)),
        ],
        out_specs=pl.BlockSpec((NBLK, D), lambda i: (i, 0)),
        out_shape=jax.ShapeDtypeStruct((N, D), jnp.float32),
    )


_sc_parts = _make_sc_parts()
_sc_cnt = _make_sc_cnt()
_tc_gelu = _make_tc(True)
_tc_plain = _make_tc(False)


def _parts_of(x):
    return x.reshape(N, NPARTS, L).transpose(1, 0, 2).reshape(NPARTS * N, L)


def kernel(embs, edge_index, rel_type, batch_size,
           weight1, root1, bias1, weight2, root2, bias2):
    src = edge_index[0]
    dst = edge_index[1]
    zeros_small = jnp.zeros((ZROWS, L), jnp.float32)

    xp = _parts_of(embs)
    cnt = _sc_cnt(dst, rel_type, zeros_small)
    a1 = _sc_parts(xp, src, dst, rel_type, zeros_small)
    cnt2 = cnt[:, :, 0].reshape(NC, R, N).transpose(2, 0, 1).reshape(N, NC * R)
    h = _tc_gelu(embs, a1, cnt2, weight1, root1, bias1.reshape(1, D))

    hp = _parts_of(h)
    a2 = _sc_parts(hp, src, dst, rel_type, zeros_small)
    out = _tc_plain(h, a2, cnt2, weight2, root2, bias2.reshape(1, D))
    return out


# speedup vs baseline: 9.6035x; 1.1571x over previous
import functools

import jax
import jax.numpy as jnp
from jax import lax
from jax.experimental import pallas as pl
from jax.experimental.pallas import tpu as pltpu
from jax.experimental.pallas import tpu_sc as plsc

N = 10000
E = 320000
D = 128
R = 8

NC = 2
NS = 16
L = 16
NPARTS = D // L
PPC = NPARTS // NC
SHARD = E // NS
EB = 128
NB = (SHARD + EB - 1) // EB
NB2 = NB + 1
PAD = NB2 * EB
RN = R * N
DUMMY = RN
ACC_ROWS = RN + EB
TPR = RN // NS
ZROWS = 125
CH = 512


def _precompute_aidx(dst_hbm, rel_hbm, dbuf, rbuf, aidx_st, t, lanes):
    base = t * SHARD

    def chunk(ci, carry):
        pltpu.sync_copy(dst_hbm.at[pl.ds(base + ci * CH, CH)], dbuf)
        pltpu.sync_copy(rel_hbm.at[pl.ds(base + ci * CH, CH)], rbuf)

        def vec(k, carry2):
            off = k * L
            d16 = dbuf[pl.ds(off, L)]
            r16 = rbuf[pl.ds(off, L)]
            aidx_st[pl.ds(ci * CH + off, L)] = r16 * N + d16
            return carry2

        lax.fori_loop(0, CH // L, vec, 0)
        return carry

    nfull = SHARD // CH
    lax.fori_loop(0, nfull, chunk, 0)
    tail = SHARD - nfull * CH
    pltpu.sync_copy(dst_hbm.at[pl.ds(base + nfull * CH, tail)],
                    dbuf.at[pl.ds(0, tail)])
    pltpu.sync_copy(rel_hbm.at[pl.ds(base + nfull * CH, tail)],
                    rbuf.at[pl.ds(0, tail)])

    def tailvec(k, carry2):
        off = k * L
        eid = nfull * CH + off + lanes
        valid = eid < SHARD
        d16 = dbuf[pl.ds(off, L)]
        r16 = rbuf[pl.ds(off, L)]
        aidx_st[pl.ds(nfull * CH + off, L)] = jnp.where(
            valid, r16 * N + d16, jnp.int32(DUMMY))
        return carry2

    lax.fori_loop(0, (PAD - nfull * CH) // L, tailvec, 0)


def _zero_acc(zbuf, acc, t, sem):
    nz = TPR // ZROWS
    for w in range(0, nz, 8):
        descs = [
            pltpu.async_copy(
                zbuf, acc.at[pl.ds(t * TPR + i * ZROWS, ZROWS)], sem)
            for i in range(w, min(w + 8, nz))
        ]
        for d in descs:
            d.wait()


def _sc_parts_body(xp_hbm, src_hbm, dst_hbm, rel_hbm, zeros_hbm,
                   a_hbm,
                   src_st, aidx_st, dbuf, rbuf, sidx2, aidx2, rows2, zbuf,
                   sem_g, sem_s, sem_z,
                   acc):
    c = lax.axis_index("c")
    t = lax.axis_index("s")
    lanes = jnp.arange(L, dtype=jnp.int32)
    pltpu.sync_copy(src_hbm.at[pl.ds(t * SHARD, SHARD)],
                    src_st.at[pl.ds(0, SHARD)])
    _precompute_aidx(dst_hbm, rel_hbm, dbuf, rbuf, aidx_st, t, lanes)
    pltpu.sync_copy(zeros_hbm, zbuf)

    def part_pass(p, carry):
        pglob = c * PPC + p
        _zero_acc(zbuf, acc, t, sem_z)
        plsc.subcore_barrier()

        def build_idx(j, b):
            for k in range(EB // L):
                off = j * EB + k * L
                eid = off + lanes
                valid = eid < SHARD
                s16 = src_st[pl.ds(off, L)]
                sidx2[b, pl.ds(k * L, L)] = jnp.where(valid, s16 + pglob * N,
                                                      jnp.int32(0))
                aidx2[b, pl.ds(k * L, L)] = aidx_st[pl.ds(off, L)]

        def edge_pair(g, carry2):
            ja = 2 * g
            jb = 2 * g + 1
            build_idx(ja, 0)
            ga = pltpu.async_copy(xp_hbm.at[sidx2.at[0]], rows2.at[0], sem_g)
            build_idx(jb, 1)
            gb = pltpu.async_copy(xp_hbm.at[sidx2.at[1]], rows2.at[1], sem_g)
            ga.wait()
            sa = pltpu.async_copy(rows2.at[0], acc.at[aidx2.at[0]], sem_s,
                                  add=True)
            gb.wait()
            sb = pltpu.async_copy(rows2.at[1], acc.at[aidx2.at[1]], sem_s,
                                  add=True)
            sa.wait()
            sb.wait()
            return carry2

        lax.fori_loop(0, NB2 // 2, edge_pair, 0)
        plsc.subcore_barrier()
        r_own = t // 2
        n_own = (t % 2) * TPR
        pltpu.sync_copy(acc.at[pl.ds(t * TPR, TPR)],
                        a_hbm.at[pl.ds(n_own, TPR), r_own,
                                 pl.ds(pglob * L, L)])
        return carry

    lax.fori_loop(0, PPC, part_pass, 0)


def _sc_cnt_body(dst_hbm, rel_hbm, zeros_hbm,
                 cnt_hbm,
                 aidx_st, dbuf, rbuf, aidx_v, ones_v, zbuf, sem_z, acc):
    c = lax.axis_index("c")
    t = lax.axis_index("s")
    lanes = jnp.arange(L, dtype=jnp.int32)
    _precompute_aidx(dst_hbm, rel_hbm, dbuf, rbuf, aidx_st, t, lanes)
    pltpu.sync_copy(zeros_hbm, zbuf)
    for k in range(EB // L):
        ones_v[pl.ds(k * L, L), :] = jnp.ones((L, L), jnp.float32)

    _zero_acc(zbuf, acc, t, sem_z)
    plsc.subcore_barrier()

    def cnt_step(j, carry):
        for k in range(EB // L):
            aidx_v[pl.ds(k * L, L)] = aidx_st[pl.ds(j * EB + k * L, L)]
        pltpu.sync_copy(ones_v, acc.at[aidx_v], add=True)
        return carry

    half = (NB + 1) // 2
    lax.fori_loop(c * half, jnp.minimum(NB, (c + 1) * half), cnt_step, 0)
    plsc.subcore_barrier()
    pltpu.sync_copy(acc.at[pl.ds(t * TPR, TPR)],
                    cnt_hbm.at[c, pl.ds(t * TPR, TPR)])


_SC_MESH = dict(core_axis_name="c", subcore_axis_name="s")


def _make_sc_parts():
    return pl.kernel(
        _sc_parts_body,
        out_type=jax.ShapeDtypeStruct((N, R, D), jnp.float32),
        mesh=plsc.VectorSubcoreMesh(**_SC_MESH),
        scratch_types=[
            pltpu.VMEM((PAD,), jnp.int32),
            pltpu.VMEM((PAD,), jnp.int32),
            pltpu.VMEM((CH,), jnp.int32),
            pltpu.VMEM((CH,), jnp.int32),
            pltpu.VMEM((2, EB), jnp.int32),
            pltpu.VMEM((2, EB), jnp.int32),
            pltpu.VMEM((2, EB, L), jnp.float32),
            pltpu.VMEM((ZROWS, L), jnp.float32),
            pltpu.SemaphoreType.DMA,
            pltpu.SemaphoreType.DMA,
            pltpu.SemaphoreType.DMA,
            pltpu.VMEM_SHARED((ACC_ROWS, L), jnp.float32),
        ],
        compiler_params=pltpu.CompilerParams(use_tc_tiling_on_sc=False),
    )


def _make_sc_cnt():
    return pl.kernel(
        _sc_cnt_body,
        out_type=jax.ShapeDtypeStruct((NC, RN, L), jnp.float32),
        mesh=plsc.VectorSubcoreMesh(**_SC_MESH),
        scratch_types=[
            pltpu.VMEM((PAD,), jnp.int32),
            pltpu.VMEM((CH,), jnp.int32),
            pltpu.VMEM((CH,), jnp.int32),
            pltpu.VMEM((EB,), jnp.int32),
            pltpu.VMEM((EB, L), jnp.float32),
            pltpu.VMEM((ZROWS, L), jnp.float32),
            pltpu.SemaphoreType.DMA,
            pltpu.VMEM_SHARED((ACC_ROWS, L), jnp.float32),
        ],
        compiler_params=pltpu.CompilerParams(use_tc_tiling_on_sc=False),
    )


def _tc_body(apply_gelu, x_ref, ap_ref, cnt_ref, w_ref, root_ref, bias_ref, o_ref):
    x = x_ref[...]
    acc = jnp.dot(x, root_ref[...], preferred_element_type=jnp.float32)
    acc = acc + bias_ref[...]
    cnt = cnt_ref[:, :R] + cnt_ref[:, R:]
    inv = 1.0 / jnp.clip(cnt, 1.0, None)
    for r in range(R):
        a = ap_ref[:, r, :] * inv[:, r][:, None]
        acc = acc + jnp.dot(a, w_ref[r], preferred_element_type=jnp.float32)
    if apply_gelu:
        acc = 0.5 * acc * (1.0 + lax.erf(acc * (2.0 ** -0.5)))
    o_ref[...] = acc


NBLK = 1000


def _make_tc(apply_gelu):
    return pl.pallas_call(
        functools.partial(_tc_body, apply_gelu),
        grid=(N // NBLK,),
        in_specs=[
            pl.BlockSpec((NBLK, D), lambda i: (i, 0)),
            pl.BlockSpec((NBLK, R, D), lambda i: (i, 0, 0)),
            pl.BlockSpec((NBLK, NC * R), lambda i: (i, 0)),
            pl.BlockSpec((R, D, D), lambda i: (0, 0, 0)),
            pl.BlockSpec((D, D), lambda i: (0, 0)),
            pl.BlockSpec((1, D), lambda i: (0, 0)),
        ],
        out_specs=pl.BlockSpec((NBLK, D), lambda i: (i, 0)),
        out_shape=jax.ShapeDtypeStruct((N, D), jnp.float32),
    )


_sc_parts = _make_sc_parts()
_sc_cnt = _make_sc_cnt()
_tc_gelu = _make_tc(True)
_tc_plain = _make_tc(False)


def _parts_of(x):
    return x.reshape(N, NPARTS, L).transpose(1, 0, 2).reshape(NPARTS * N, L)


def kernel(embs, edge_index, rel_type, batch_size,
           weight1, root1, bias1, weight2, root2, bias2):
    src = edge_index[0]
    dst = edge_index[1]
    zeros_small = jnp.zeros((ZROWS, L), jnp.float32)

    xp = _parts_of(embs)
    cnt = _sc_cnt(dst, rel_type, zeros_small)
    a1 = _sc_parts(xp, src, dst, rel_type, zeros_small)
    cnt2 = cnt[:, :, 0].reshape(NC, R, N).transpose(2, 0, 1).reshape(N, NC * R)
    h = _tc_gelu(embs, a1, cnt2, weight1, root1, bias1.reshape(1, D))

    hp = _parts_of(h)
    a2 = _sc_parts(hp, src, dst, rel_type, zeros_small)
    out = _tc_plain(h, a2, cnt2, weight2, root2, bias2.reshape(1, D))
    return out


# speedup vs baseline: 12.6629x; 1.3186x over previous
import functools

import jax
import jax.numpy as jnp
from jax import lax
from jax.experimental import pallas as pl
from jax.experimental.pallas import tpu as pltpu
from jax.experimental.pallas import tpu_sc as plsc

N = 10000
E = 320000
D = 128
R = 8

NC = 2
NS = 16
L = 16
NPARTS = D // L
PPC = NPARTS // NC
SHARD = E // NS
EB = 128
NB = (SHARD + EB - 1) // EB
NB2 = NB + 1
PAD = NB2 * EB
RN = R * N
DUMMY = RN
ACC_ROWS = RN + EB
TPR = RN // NS
ZROWS = 200
CH = 512

EROWS = E // EB
TROWS = EROWS // NS
CROWS = 16
GR = 4


def _precompute_aidx(dst_hbm, rel_hbm, dbuf, rbuf, aidx_st, t, lanes):
    base = t * SHARD

    def chunk(ci, carry):
        pltpu.sync_copy(dst_hbm.at[pl.ds(base + ci * CH, CH)], dbuf)
        pltpu.sync_copy(rel_hbm.at[pl.ds(base + ci * CH, CH)], rbuf)

        def vec(k, carry2):
            off = k * L
            d16 = dbuf[pl.ds(off, L)]
            r16 = rbuf[pl.ds(off, L)]
            aidx_st[pl.ds(ci * CH + off, L)] = r16 * N + d16
            return carry2

        lax.fori_loop(0, CH // L, vec, 0)
        return carry

    nfull = SHARD // CH
    lax.fori_loop(0, nfull, chunk, 0)
    tail = SHARD - nfull * CH
    pltpu.sync_copy(dst_hbm.at[pl.ds(base + nfull * CH, tail)],
                    dbuf.at[pl.ds(0, tail)])
    pltpu.sync_copy(rel_hbm.at[pl.ds(base + nfull * CH, tail)],
                    rbuf.at[pl.ds(0, tail)])

    def tailvec(k, carry2):
        off = k * L
        eid = nfull * CH + off + lanes
        valid = eid < SHARD
        d16 = dbuf[pl.ds(off, L)]
        r16 = rbuf[pl.ds(off, L)]
        aidx_st[pl.ds(nfull * CH + off, L)] = jnp.where(
            valid, r16 * N + d16, jnp.int32(DUMMY))
        return carry2

    lax.fori_loop(0, (PAD - nfull * CH) // L, tailvec, 0)


def _zero_acc(zbuf, acc, t, sem):
    nz = TPR // ZROWS
    for w in range(0, nz, 8):
        descs = [
            pltpu.async_copy(
                zbuf, acc.at[pl.ds(t * TPR + i * ZROWS, ZROWS)], sem)
            for i in range(w, min(w + 8, nz))
        ]
        for d in descs:
            d.wait()


def _sc_parts_body(xp_hbm, src2_hbm, dst2_hbm, rel2_hbm, zeros_hbm,
                   a_hbm,
                   packed_st, sbuf, dbuf, rbuf, sidx2, aidx2, rows2, zbuf,
                   sem_g, sem_s, sem_z,
                   acc):
    c = lax.axis_index("c")
    t = lax.axis_index("s")
    row_lo = t * TROWS

    def pack_chunk(rbase, nr):
        pltpu.sync_copy(src2_hbm.at[pl.ds(row_lo + rbase, nr)],
                        sbuf.at[pl.ds(0, nr)])
        pltpu.sync_copy(dst2_hbm.at[pl.ds(row_lo + rbase, nr)],
                        dbuf.at[pl.ds(0, nr)])
        pltpu.sync_copy(rel2_hbm.at[pl.ds(row_lo + rbase, nr)],
                        rbuf.at[pl.ds(0, nr)])
        for rr in range(nr):
            for k in range(EB // L):
                s16 = sbuf[rr, pl.ds(k * L, L)]
                d16 = dbuf[rr, pl.ds(k * L, L)]
                r16 = rbuf[rr, pl.ds(k * L, L)]
                packed_st[pl.ds((rbase + rr) * EB + k * L, L)] = (
                    ((r16 * N + d16) << 14) | s16)

    def pack_chunk_loop(ci, carry):
        pltpu.sync_copy(src2_hbm.at[pl.ds(row_lo + ci * CROWS, CROWS)],
                        sbuf)
        pltpu.sync_copy(dst2_hbm.at[pl.ds(row_lo + ci * CROWS, CROWS)],
                        dbuf)
        pltpu.sync_copy(rel2_hbm.at[pl.ds(row_lo + ci * CROWS, CROWS)],
                        rbuf)
        for rr in range(CROWS):
            for k in range(EB // L):
                s16 = sbuf[rr, pl.ds(k * L, L)]
                d16 = dbuf[rr, pl.ds(k * L, L)]
                r16 = rbuf[rr, pl.ds(k * L, L)]
                packed_st[pl.ds((ci * CROWS + rr) * EB + k * L, L)] = (
                    ((r16 * N + d16) << 14) | s16)
        return carry

    lax.fori_loop(0, 9, pack_chunk_loop, 0)

    @pl.when(t == NS - 1)
    def _():
        pack_chunk(9 * CROWS, CROWS)

    @pl.when(t < NS - 1)
    def _():
        pack_chunk(9 * CROWS, TROWS - 9 * CROWS)

    pltpu.sync_copy(zeros_hbm, zbuf)

    def part_pass(p, carry):
        pglob = c * PPC + p
        _zero_acc(zbuf, acc, t, sem_z)
        plsc.subcore_barrier()

        def build_idx(j, b):
            for k in range(EB // L):
                pk = packed_st[pl.ds(j * EB + k * L, L)]
                sidx2[b, pl.ds(k * L, L)] = (pk & 16383) + pglob * N
                aidx2[b, pl.ds(k * L, L)] = pk >> 14

        def fire_gather(b):
            return pltpu.async_copy(xp_hbm.at[sidx2.at[b]], rows2.at[b],
                                    sem_g)

        def fire_scatter(b):
            return pltpu.async_copy(rows2.at[b], acc.at[aidx2.at[b]], sem_s,
                                    add=True)

        def drain_scatter(b):
            pltpu.make_async_copy(rows2.at[b], acc.at[aidx2.at[b]],
                                  sem_s).wait()

        def edge_pair(g, parity, with_drain):
            b0 = 2 * parity
            b1 = b0 + 1
            if with_drain:
                drain_scatter(b0)
                drain_scatter(b1)
            build_idx(2 * g, b0)
            ga = fire_gather(b0)
            build_idx(2 * g + 1, b1)
            gb = fire_gather(b1)
            ga.wait()
            fire_scatter(b0)
            gb.wait()
            fire_scatter(b1)

        edge_pair(0, 0, False)
        edge_pair(1, 1, False)

        def ring_step(g2, carry2):
            edge_pair(2 * g2, 0, True)
            edge_pair(2 * g2 + 1, 1, True)
            return carry2

        lax.fori_loop(1, TROWS // 4, ring_step, 0)

        @pl.when(t == NS - 1)
        def _():
            edge_pair(TROWS // 2, 0, True)
            edge_pair(TROWS // 2 + 1, 1, True)

        for b in range(4):
            drain_scatter(b)

        plsc.subcore_barrier()
        r_own = t // 2
        n_own = (t % 2) * TPR
        pltpu.sync_copy(acc.at[pl.ds(t * TPR, TPR)],
                        a_hbm.at[pl.ds(n_own, TPR), r_own,
                                 pl.ds(pglob * L, L)])
        return carry

    lax.fori_loop(0, PPC, part_pass, 0)


def _sc_cnt_body(dst_hbm, rel_hbm, zeros_hbm,
                 cnt_hbm,
                 aidx_st, dbuf, rbuf, aidx_v, ones_v, zbuf, sem_z, acc):
    c = lax.axis_index("c")
    t = lax.axis_index("s")
    lanes = jnp.arange(L, dtype=jnp.int32)
    _precompute_aidx(dst_hbm, rel_hbm, dbuf, rbuf, aidx_st, t, lanes)
    pltpu.sync_copy(zeros_hbm, zbuf)
    for k in range(EB // L):
        ones_v[pl.ds(k * L, L), :] = jnp.ones((L, L), jnp.float32)

    _zero_acc(zbuf, acc, t, sem_z)
    plsc.subcore_barrier()

    def cnt_step(j, carry):
        for k in range(EB // L):
            aidx_v[pl.ds(k * L, L)] = aidx_st[pl.ds(j * EB + k * L, L)]
        pltpu.sync_copy(ones_v, acc.at[aidx_v], add=True)
        return carry

    half = (NB + 1) // 2
    lax.fori_loop(c * half, jnp.minimum(NB, (c + 1) * half), cnt_step, 0)
    plsc.subcore_barrier()
    pltpu.sync_copy(acc.at[pl.ds(t * TPR, TPR)],
                    cnt_hbm.at[c, pl.ds(t * TPR, TPR)])


_SC_MESH = dict(core_axis_name="c", subcore_axis_name="s")


def _make_sc_parts():
    return pl.kernel(
        _sc_parts_body,
        out_type=jax.ShapeDtypeStruct((N, R, D), jnp.float32),
        mesh=plsc.VectorSubcoreMesh(**_SC_MESH),
        scratch_types=[
            pltpu.VMEM(((TROWS + 4) * EB,), jnp.int32),
            pltpu.VMEM((CROWS, EB), jnp.int32),
            pltpu.VMEM((CROWS, EB), jnp.int32),
            pltpu.VMEM((CROWS, EB), jnp.int32),
            pltpu.VMEM((4, EB), jnp.int32),
            pltpu.VMEM((4, EB), jnp.int32),
            pltpu.VMEM((4, EB, L), jnp.float32),
            pltpu.VMEM((ZROWS, L), jnp.float32),
            pltpu.SemaphoreType.DMA,
            pltpu.SemaphoreType.DMA,
            pltpu.SemaphoreType.DMA,
            pltpu.VMEM_SHARED((ACC_ROWS, L), jnp.float32),
        ],
        compiler_params=pltpu.CompilerParams(use_tc_tiling_on_sc=False),
    )


def _make_sc_cnt():
    return pl.kernel(
        _sc_cnt_body,
        out_type=jax.ShapeDtypeStruct((NC, RN, L), jnp.float32),
        mesh=plsc.VectorSubcoreMesh(**_SC_MESH),
        scratch_types=[
            pltpu.VMEM((PAD,), jnp.int32),
            pltpu.VMEM((CH,), jnp.int32),
            pltpu.VMEM((CH,), jnp.int32),
            pltpu.VMEM((EB,), jnp.int32),
            pltpu.VMEM((EB, L), jnp.float32),
            pltpu.VMEM((ZROWS, L), jnp.float32),
            pltpu.SemaphoreType.DMA,
            pltpu.VMEM_SHARED((ACC_ROWS, L), jnp.float32),
        ],
        compiler_params=pltpu.CompilerParams(use_tc_tiling_on_sc=False),
    )


def _tc_body(apply_gelu, x_ref, ap_ref, cnt_ref, w_ref, root_ref, bias_ref, o_ref):
    x = x_ref[...]
    acc = jnp.dot(x, root_ref[...], preferred_element_type=jnp.float32)
    acc = acc + bias_ref[...]
    cnt = cnt_ref[:, :R] + cnt_ref[:, R:]
    inv = 1.0 / jnp.clip(cnt, 1.0, None)
    for r in range(R):
        a = ap_ref[:, r, :] * inv[:, r][:, None]
        acc = acc + jnp.dot(a, w_ref[r], preferred_element_type=jnp.float32)
    if apply_gelu:
        acc = 0.5 * acc * (1.0 + lax.erf(acc * (2.0 ** -0.5)))
    o_ref[...] = acc


NBLK = 1000


def _make_tc(apply_gelu):
    return pl.pallas_call(
        functools.partial(_tc_body, apply_gelu),
        grid=(N // NBLK,),
        in_specs=[
            pl.BlockSpec((NBLK, D), lambda i: (i, 0)),
            pl.BlockSpec((NBLK, R, D), lambda i: (i, 0, 0)),
            pl.BlockSpec((NBLK, NC * R), lambda i: (i, 0)),
            pl.BlockSpec((R, D, D), lambda i: (0, 0, 0)),
            pl.BlockSpec((D, D), lambda i: (0, 0)),
            pl.BlockSpec((1, D), lambda i: (0, 0)),
        ],
        out_specs=pl.BlockSpec((NBLK, D), lambda i: (i, 0)),
        out_shape=jax.ShapeDtypeStruct((N, D), jnp.float32),
    )


_sc_parts = _make_sc_parts()
_sc_cnt = _make_sc_cnt()
_tc_gelu = _make_tc(True)
_tc_plain = _make_tc(False)


def _parts_of(x):
    return x.reshape(N, NPARTS, L).transpose(1, 0, 2).reshape(NPARTS * N, L)


def kernel(embs, edge_index, rel_type, batch_size,
           weight1, root1, bias1, weight2, root2, bias2):
    src = edge_index[0]
    dst = edge_index[1]
    zeros_small = jnp.zeros((ZROWS, L), jnp.float32)

    src2 = src.reshape(EROWS, EB)
    dst2 = dst.reshape(EROWS, EB)
    rel2 = rel_type.reshape(EROWS, EB)

    xp = _parts_of(embs)
    cnt = _sc_cnt(dst, rel_type, zeros_small)
    a1 = _sc_parts(xp, src2, dst2, rel2, zeros_small)
    cnt2 = cnt[:, :, 0].reshape(NC, R, N).transpose(2, 0, 1).reshape(N, NC * R)
    h = _tc_gelu(embs, a1, cnt2, weight1, root1, bias1.reshape(1, D))

    hp = _parts_of(h)
    a2 = _sc_parts(hp, src2, dst2, rel2, zeros_small)
    out = _tc_plain(h, a2, cnt2, weight2, root2, bias2.reshape(1, D))
    return out
